# baseline (speedup 1.0000x reference)
import jax
import jax.numpy as jnp
from jax import lax
from jax.experimental import pallas as pl
from jax.experimental.pallas import tpu as pltpu
from jax.experimental.pallas import tpu_sc as plsc

N_FIELDS = 13
VOCAB = 100000
DIM = 16
BATCH = 16384

NC, NS = 2, 16
NW = NC * NS
BPW = BATCH // NW
NB = 128
NCH = BPW // NB
K = N_FIELDS * NB
KP = K + 16


def _body(x_hbm, ut_hbm, it_hbm, out_hbm, xv, uidx, iidx,
          uo0, uo1, io0, io1, ur0, ur1, ir0, ir1, sg0, sg1, sw0, sw1):
    wid = lax.axis_index("s") * NC + lax.axis_index("c")
    uo, io = [uo0, uo1], [io0, io1]
    urows, irows = [ur0, ur1], [ir0, ir1]
    sg, sw = [sg0, sg1], [sw0, sw1]
    lanes = lax.iota(jnp.int32, 16)
    offv = lanes * VOCAB

    prev_writes = [None, None]
    for ch in range(NCH):
        bsel = ch & 1
        base = wid * BPW + ch * NB

        pltpu.sync_copy(x_hbm.at[pl.ds(base * 26, NB * 26)],
                        xv.at[pl.ds(0, NB * 26)])

        uo_b, io_b = uo[bsel], io[bsel]

        def build(b, _):
            p = b * N_FIELDS
            q = b * 26
            ru = xv[pl.ds(q, 16)]
            ri = xv[pl.ds(q + 13, 16)]
            uidx[pl.ds(p, 16)] = ru + offv
            iidx[pl.ds(p, 16)] = ri + offv
            ob = (base + b) * 26
            uo_b[pl.ds(p, 16)] = lanes + ob
            io_b[pl.ds(p, 16)] = lanes + (ob + N_FIELDS)
            return 0

        lax.fori_loop(0, NB, build, 0)

        if prev_writes[bsel] is not None:
            for w in prev_writes[bsel]:
                w.wait()

        g_u = pltpu.async_copy(
            ut_hbm.at[uidx.at[pl.ds(0, K)]], urows[bsel], sg[bsel])
        g_i = pltpu.async_copy(
            it_hbm.at[iidx.at[pl.ds(0, K)]], irows[bsel], sg[bsel])
        g_u.wait()
        g_i.wait()

        w_u = pltpu.async_copy(
            urows[bsel], out_hbm.at[uo_b.at[pl.ds(0, K)]], sw[bsel])
        w_i = pltpu.async_copy(
            irows[bsel], out_hbm.at[io_b.at[pl.ds(0, K)]], sw[bsel])
        prev_writes[bsel] = (w_u, w_i)

    for pw in prev_writes:
        if pw is not None:
            for w in pw:
                w.wait()


@jax.jit
def kernel(x, user_tables, item_tables):
    uflat = user_tables.reshape(N_FIELDS * VOCAB, DIM)
    iflat = item_tables.reshape(N_FIELDS * VOCAB, DIM)
    mesh = plsc.VectorSubcoreMesh(
        core_axis_name="c", subcore_axis_name="s",
        num_cores=NC, num_subcores=NS)
    outf = pl.kernel(
        _body,
        out_type=jax.ShapeDtypeStruct((BATCH * 2 * N_FIELDS, DIM), jnp.float32),
        mesh=mesh,
        compiler_params=pltpu.CompilerParams(use_tc_tiling_on_sc=False),
        scratch_types=[
            pltpu.VMEM((NB * 26 + 16,), jnp.int32),
            pltpu.VMEM((KP,), jnp.int32),
            pltpu.VMEM((KP,), jnp.int32),
            pltpu.VMEM((KP,), jnp.int32),
            pltpu.VMEM((KP,), jnp.int32),
            pltpu.VMEM((KP,), jnp.int32),
            pltpu.VMEM((KP,), jnp.int32),
            pltpu.VMEM((K, DIM), jnp.float32),
            pltpu.VMEM((K, DIM), jnp.float32),
            pltpu.VMEM((K, DIM), jnp.float32),
            pltpu.VMEM((K, DIM), jnp.float32),
            pltpu.SemaphoreType.DMA,
            pltpu.SemaphoreType.DMA,
            pltpu.SemaphoreType.DMA,
            pltpu.SemaphoreType.DMA,
        ],
    )(x.astype(jnp.int32).reshape(BATCH * 2 * N_FIELDS), uflat, iflat)
    return outf.reshape(BATCH, 2 * N_FIELDS * DIM)


# speedup vs baseline: 1.0033x; 1.0033x over previous
import jax
import jax.numpy as jnp
from jax import lax
from jax.experimental import pallas as pl
from jax.experimental.pallas import tpu as pltpu
from jax.experimental.pallas import tpu_sc as plsc

N_FIELDS = 13
VOCAB = 100000
DIM = 16
BATCH = 16384

NC, NS = 2, 16
NW = NC * NS
BPW = BATCH // NW
NT = 2 * N_FIELDS
NBUF = 4
PRE = 2


def _body(xt_hbm, ut_hbm, it_hbm, out_hbm, xv, r0, r1, r2, r3,
          sg0, sg1, sg2, sg3, sw0, sw1, sw2, sw3):
    wid = lax.axis_index("s") * NC + lax.axis_index("c")
    base = wid * BPW
    rows = [r0, r1, r2, r3]
    sg = [sg0, sg1, sg2, sg3]
    sw = [sw0, sw1, sw2, sw3]

    pltpu.sync_copy(xt_hbm.at[:, pl.ds(base, BPW)], xv)

    gs = [None] * NBUF
    ws = [None] * NBUF

    def issue_gather(t):
        b = t % NBUF
        if ws[b] is not None:
            ws[b].wait()
        tab = ut_hbm if t < N_FIELDS else it_hbm
        f = t if t < N_FIELDS else t - N_FIELDS
        gs[b] = pltpu.async_copy(tab.at[f].at[xv.at[t]], rows[b], sg[b])

    for t in range(PRE):
        issue_gather(t)
    for t in range(NT):
        b = t % NBUF
        if t + PRE < NT:
            issue_gather(t + PRE)
        gs[b].wait()
        ws[b] = pltpu.async_copy(
            rows[b], out_hbm.at[pl.ds(base, BPW), pl.ds(t * DIM, DIM)], sw[b])
    for b in range(NBUF):
        if ws[b] is not None:
            ws[b].wait()


@jax.jit
def kernel(x, user_tables, item_tables):
    xt = x.astype(jnp.int32).T
    mesh = plsc.VectorSubcoreMesh(
        core_axis_name="c", subcore_axis_name="s",
        num_cores=NC, num_subcores=NS)
    return pl.kernel(
        _body,
        out_type=jax.ShapeDtypeStruct((BATCH, NT * DIM), jnp.float32),
        mesh=mesh,
        compiler_params=pltpu.CompilerParams(use_tc_tiling_on_sc=False),
        scratch_types=[
            pltpu.VMEM((NT, BPW), jnp.int32),
            pltpu.VMEM((BPW, DIM), jnp.float32),
            pltpu.VMEM((BPW, DIM), jnp.float32),
            pltpu.VMEM((BPW, DIM), jnp.float32),
            pltpu.VMEM((BPW, DIM), jnp.float32),
            pltpu.SemaphoreType.DMA,
            pltpu.SemaphoreType.DMA,
            pltpu.SemaphoreType.DMA,
            pltpu.SemaphoreType.DMA,
            pltpu.SemaphoreType.DMA,
            pltpu.SemaphoreType.DMA,
            pltpu.SemaphoreType.DMA,
            pltpu.SemaphoreType.DMA,
        ],
    )(xt, user_tables, item_tables)
